# baseline (device time: 282980 ns/iter reference)
import jax
import jax.numpy as jnp
from jax import lax
from jax.experimental import pallas as pl
from jax.experimental.pallas import tpu as pltpu

B, H, D, BS = 32, 16, 128, 32
NEG = -1e30


def _compute_body(q_ref, k_ref, v_ref, bt_ref, lens_ref, o_ref, st_ref):
    c = pl.program_id(1)
    n_pages = k_ref.shape[0]
    t = n_pages * BS

    @pl.when(c == 0)
    def _():
        o_ref[...] = jnp.zeros_like(o_ref)
        st_ref[...] = jnp.zeros_like(st_ref)
        st_ref[:, 0:1] = jnp.full((B, 1), NEG, jnp.float32)

    z = lax.axis_index("z")
    base = z * 256 + c * n_pages

    bt = bt_ref[...]
    nb = bt.shape[1]
    jidx = lax.broadcasted_iota(jnp.int32, bt.shape, 1)
    validf = (jidx < lens_ref[...]).astype(jnp.float32)
    bt3 = lax.broadcast_in_dim(bt, (B, n_pages, nb), (0, 2))
    valid3 = lax.broadcast_in_dim(validf, (B, n_pages, nb), (0, 2))
    pidx3 = lax.broadcasted_iota(jnp.int32, (B, n_pages, nb), 1) + base
    eqf = (bt3 == pidx3).astype(jnp.float32) * valid3
    counts = jnp.sum(eqf, axis=2)
    w = jnp.broadcast_to(counts[:, :, None], (B, n_pages, BS)).reshape(B, t)

    q = (q_ref[...].reshape(B, D) * (D ** -0.5)).astype(jnp.bfloat16)
    k = k_ref[...].reshape(t, D).astype(jnp.bfloat16)
    s = lax.dot_general(
        q, k,
        dimension_numbers=(((1,), (1,)), ((), ())),
        preferred_element_type=jnp.float32,
    )

    s_masked = jnp.where(w > 0.0, s, NEG)
    m_c = jnp.max(s_masked, axis=1, keepdims=True)
    m_old = st_ref[:, 0:1]
    l_old = st_ref[:, 1:2]
    m_new = jnp.maximum(m_old, m_c)
    p = w * jnp.exp(jnp.minimum(s - m_new, 0.0))
    scale = jnp.exp(m_old - m_new)

    st_ref[:, 0:1] = m_new
    st_ref[:, 1:2] = l_old * scale + jnp.sum(p, axis=1, keepdims=True)
    v = v_ref[...].reshape(t, D).astype(jnp.bfloat16)
    o_c = lax.dot_general(
        p.astype(jnp.bfloat16), v,
        dimension_numbers=(((1,), (0,)), ((), ())),
        preferred_element_type=jnp.float32,
    )
    o_ref[...] = o_ref[...] * scale + o_c


def _exchange_body(o_ref, s_ref, out_ref, comm_o, comm_s, send_sems, recv_sems):
    x = lax.axis_index("x")
    y = lax.axis_index("y")
    z = lax.axis_index("z")
    nbr = (x, y, 1 - z)

    barrier_sem = pltpu.get_barrier_semaphore()
    pl.semaphore_signal(
        barrier_sem, inc=1, device_id=nbr,
        device_id_type=pl.DeviceIdType.MESH,
    )
    pl.semaphore_wait(barrier_sem, 1)

    rdma_o = pltpu.make_async_remote_copy(
        src_ref=o_ref, dst_ref=comm_o,
        send_sem=send_sems.at[0], recv_sem=recv_sems.at[0],
        device_id=nbr, device_id_type=pl.DeviceIdType.MESH,
    )
    rdma_s = pltpu.make_async_remote_copy(
        src_ref=s_ref, dst_ref=comm_s,
        send_sem=send_sems.at[1], recv_sem=recv_sems.at[1],
        device_id=nbr, device_id_type=pl.DeviceIdType.MESH,
    )
    rdma_o.start()
    rdma_s.start()
    rdma_o.wait()
    rdma_s.wait()

    m0 = s_ref[:, 0:1]
    l0 = s_ref[:, 1:2]
    m1 = comm_s[:, 0:1]
    l1 = comm_s[:, 1:2]
    mg = jnp.maximum(m0, m1)
    a0 = jnp.exp(m0 - mg)
    a1 = jnp.exp(m1 - mg)
    lg = l0 * a0 + l1 * a1
    o = o_ref[...] * (a0 / lg) + comm_o[...] * (a1 / lg)
    out_ref[...] = jnp.transpose(o.reshape(H, B, D), (1, 0, 2)) \
        .reshape(B, 1, H, D)


def kernel(Q, K, V, bt, lens):
    n_local_pages = K.shape[0]
    chunk_pages = 32
    n_chunks = n_local_pages // chunk_pages

    qt = jnp.transpose(Q.reshape(B, H, D), (1, 0, 2))
    k2 = K.reshape(n_local_pages, BS, H * D)
    v2 = V.reshape(n_local_pages, BS, H * D)
    lens2 = lens.reshape(B, 1)

    o_un, stats = pl.pallas_call(
        _compute_body,
        grid=(H, n_chunks),
        in_specs=[
            pl.BlockSpec((1, B, D), lambda h, c: (h, 0, 0)),
            pl.BlockSpec((chunk_pages, BS, D), lambda h, c: (c, 0, h)),
            pl.BlockSpec((chunk_pages, BS, D), lambda h, c: (c, 0, h)),
            pl.BlockSpec(bt.shape, lambda h, c: (0, 0)),
            pl.BlockSpec((B, 1), lambda h, c: (0, 0)),
        ],
        out_specs=[
            pl.BlockSpec((B, D), lambda h, c: (h, 0)),
            pl.BlockSpec((B, 8), lambda h, c: (h, 0)),
        ],
        out_shape=[
            jax.ShapeDtypeStruct((H * B, D), jnp.float32),
            jax.ShapeDtypeStruct((H * B, 8), jnp.float32),
        ],
    )(qt, k2, v2, bt, lens2)

    return pl.pallas_call(
        _exchange_body,
        out_shape=jax.ShapeDtypeStruct((B, 1, H, D), jnp.float32),
        in_specs=[
            pl.BlockSpec(memory_space=pltpu.VMEM),
            pl.BlockSpec(memory_space=pltpu.VMEM),
        ],
        out_specs=pl.BlockSpec(memory_space=pltpu.VMEM),
        scratch_shapes=[
            pltpu.VMEM((H * B, D), jnp.float32),
            pltpu.VMEM((H * B, 8), jnp.float32),
            pltpu.SemaphoreType.DMA((2,)),
            pltpu.SemaphoreType.DMA((2,)),
        ],
        compiler_params=pltpu.CompilerParams(collective_id=0),
    )(o_un, stats)


# device time: 240339 ns/iter; 1.1774x vs baseline; 1.1774x over previous
import jax
import jax.numpy as jnp
from jax import lax
from jax.experimental import pallas as pl
from jax.experimental.pallas import tpu as pltpu

B, H, D, BS = 32, 16, 128, 32
NEG = -1e30


def _compute_body(q_ref, k_ref, v_ref, bt_ref, lens_ref, o_ref, st_ref):
    c = pl.program_id(0)
    n_pages = k_ref.shape[0]
    t = n_pages * BS

    @pl.when(c == 0)
    def _():
        o_ref[...] = jnp.zeros_like(o_ref)
        st_ref[...] = jnp.zeros_like(st_ref)
        st_ref[:, 0:1] = jnp.full((H * B, 1), NEG, jnp.float32)

    z = lax.axis_index("z")
    base = z * 256 + c * n_pages

    bt = bt_ref[...]
    nb = bt.shape[1]
    jidx = lax.broadcasted_iota(jnp.int32, bt.shape, 1)
    validf = (jidx < lens_ref[...]).astype(jnp.float32)
    bt3 = lax.broadcast_in_dim(bt, (B, n_pages, nb), (0, 2))
    valid3 = lax.broadcast_in_dim(validf, (B, n_pages, nb), (0, 2))
    pidx3 = lax.broadcasted_iota(jnp.int32, (B, n_pages, nb), 1) + base
    eqf = (bt3 == pidx3).astype(jnp.float32) * valid3
    counts = jnp.sum(eqf, axis=2)
    w = jnp.broadcast_to(counts[:, :, None], (B, n_pages, BS)).reshape(B, t)
    wpos = (w > 0.0)

    q_all = q_ref[...] * (D ** -0.5)
    k_all = k_ref[...].reshape(t, H * D).astype(jnp.bfloat16)
    v_all = v_ref[...].reshape(t, H * D).astype(jnp.bfloat16)

    for h in range(H):
        sl = slice(h * D, (h + 1) * D)
        rows = slice(h * B, (h + 1) * B)
        qh = q_all[:, sl].astype(jnp.bfloat16)
        kh = k_all[:, sl]
        s = lax.dot_general(
            qh, kh,
            dimension_numbers=(((1,), (1,)), ((), ())),
            preferred_element_type=jnp.float32,
        )

        s_masked = jnp.where(wpos, s, NEG)
        m_c = jnp.max(s_masked, axis=1, keepdims=True)
        m_old = st_ref[rows, 0:1]
        l_old = st_ref[rows, 1:2]
        m_new = jnp.maximum(m_old, m_c)
        p = w * jnp.exp(jnp.minimum(s - m_new, 0.0))
        scale = jnp.exp(m_old - m_new)

        st_ref[rows, 0:1] = m_new
        st_ref[rows, 1:2] = l_old * scale + jnp.sum(p, axis=1, keepdims=True)
        o_c = lax.dot_general(
            p.astype(jnp.bfloat16), v_all[:, sl],
            dimension_numbers=(((1,), (0,)), ((), ())),
            preferred_element_type=jnp.float32,
        )
        o_ref[rows, :] = o_ref[rows, :] * scale + o_c


def _exchange_body(o_ref, s_ref, out_ref, comm_o, comm_s, send_sems, recv_sems):
    x = lax.axis_index("x")
    y = lax.axis_index("y")
    z = lax.axis_index("z")
    nbr = (x, y, 1 - z)

    barrier_sem = pltpu.get_barrier_semaphore()
    pl.semaphore_signal(
        barrier_sem, inc=1, device_id=nbr,
        device_id_type=pl.DeviceIdType.MESH,
    )
    pl.semaphore_wait(barrier_sem, 1)

    rdma_o = pltpu.make_async_remote_copy(
        src_ref=o_ref, dst_ref=comm_o,
        send_sem=send_sems.at[0], recv_sem=recv_sems.at[0],
        device_id=nbr, device_id_type=pl.DeviceIdType.MESH,
    )
    rdma_s = pltpu.make_async_remote_copy(
        src_ref=s_ref, dst_ref=comm_s,
        send_sem=send_sems.at[1], recv_sem=recv_sems.at[1],
        device_id=nbr, device_id_type=pl.DeviceIdType.MESH,
    )
    rdma_o.start()
    rdma_s.start()
    rdma_o.wait()
    rdma_s.wait()

    m0 = s_ref[:, 0:1]
    l0 = s_ref[:, 1:2]
    m1 = comm_s[:, 0:1]
    l1 = comm_s[:, 1:2]
    mg = jnp.maximum(m0, m1)
    a0 = jnp.exp(m0 - mg)
    a1 = jnp.exp(m1 - mg)
    lg = l0 * a0 + l1 * a1
    o = o_ref[...] * (a0 / lg) + comm_o[...] * (a1 / lg)
    out_ref[...] = jnp.transpose(o.reshape(H, B, D), (1, 0, 2)) \
        .reshape(B, 1, H, D)


def kernel(Q, K, V, bt, lens):
    n_local_pages = K.shape[0]
    chunk_pages = 16
    n_chunks = n_local_pages // chunk_pages

    qf = Q.reshape(B, H * D)
    k2 = K.reshape(n_local_pages, BS, H * D)
    v2 = V.reshape(n_local_pages, BS, H * D)
    lens2 = lens.reshape(B, 1)

    o_un, stats = pl.pallas_call(
        _compute_body,
        grid=(n_chunks,),
        in_specs=[
            pl.BlockSpec((B, H * D), lambda c: (0, 0)),
            pl.BlockSpec((chunk_pages, BS, H * D), lambda c: (c, 0, 0)),
            pl.BlockSpec((chunk_pages, BS, H * D), lambda c: (c, 0, 0)),
            pl.BlockSpec(bt.shape, lambda c: (0, 0)),
            pl.BlockSpec((B, 1), lambda c: (0, 0)),
        ],
        out_specs=[
            pl.BlockSpec((H * B, D), lambda c: (0, 0)),
            pl.BlockSpec((H * B, 8), lambda c: (0, 0)),
        ],
        out_shape=[
            jax.ShapeDtypeStruct((H * B, D), jnp.float32),
            jax.ShapeDtypeStruct((H * B, 8), jnp.float32),
        ],
    )(qf, k2, v2, bt, lens2)

    return pl.pallas_call(
        _exchange_body,
        out_shape=jax.ShapeDtypeStruct((B, 1, H, D), jnp.float32),
        in_specs=[
            pl.BlockSpec(memory_space=pltpu.VMEM),
            pl.BlockSpec(memory_space=pltpu.VMEM),
        ],
        out_specs=pl.BlockSpec(memory_space=pltpu.VMEM),
        scratch_shapes=[
            pltpu.VMEM((H * B, D), jnp.float32),
            pltpu.VMEM((H * B, 8), jnp.float32),
            pltpu.SemaphoreType.DMA((2,)),
            pltpu.SemaphoreType.DMA((2,)),
        ],
        compiler_params=pltpu.CompilerParams(collective_id=0),
    )(o_un, stats)


# device time: 134451 ns/iter; 2.1047x vs baseline; 1.7876x over previous
import jax
import jax.numpy as jnp
from jax import lax
from jax.experimental import pallas as pl
from jax.experimental.pallas import tpu as pltpu

B, H, D, BS = 32, 16, 128, 32
NEG = -1e30


def _compute_body(q_ref, k_ref, v_ref, bt_ref, lens_ref, o_ref, st_ref):
    c = pl.program_id(0)
    n_pages = k_ref.shape[0]
    t = n_pages * BS

    @pl.when(c == 0)
    def _():
        o_ref[...] = jnp.zeros_like(o_ref)
        st_ref[...] = jnp.zeros_like(st_ref)
        st_ref[:, 0:1] = jnp.full((H * B, 1), NEG, jnp.float32)

    z = lax.axis_index("z")
    base = z * 256 + c * n_pages

    bt = bt_ref[...]
    nb = bt.shape[1]
    jidx = lax.broadcasted_iota(jnp.int32, bt.shape, 1)
    validf = (jidx < lens_ref[...]).astype(jnp.float32)
    bt3 = lax.broadcast_in_dim(bt, (B, n_pages, nb), (0, 2))
    valid3 = lax.broadcast_in_dim(validf, (B, n_pages, nb), (0, 2))
    pidx3 = lax.broadcasted_iota(jnp.int32, (B, n_pages, nb), 1) + base
    eqf = (bt3 == pidx3).astype(jnp.float32) * valid3
    counts = jnp.sum(eqf, axis=2)
    w = jnp.broadcast_to(counts[:, :, None], (B, n_pages, BS)).reshape(B, t)
    wpos = (w > 0.0)

    q_all = q_ref[...] * (D ** -0.5)
    k_all = k_ref[...].reshape(t, H, D).astype(jnp.bfloat16)
    v_all = v_ref[...].reshape(t, H, D).astype(jnp.bfloat16)

    for h in range(H):
        sl = slice(h * D, (h + 1) * D)
        rows = slice(h * B, (h + 1) * B)
        qh = q_all[:, sl].astype(jnp.bfloat16)
        kh = k_all[:, h, :]
        s = lax.dot_general(
            qh, kh,
            dimension_numbers=(((1,), (1,)), ((), ())),
            preferred_element_type=jnp.float32,
        )

        s_masked = jnp.where(wpos, s, NEG)
        m_c = jnp.max(s_masked, axis=1, keepdims=True)
        m_old = st_ref[rows, 0:1]
        l_old = st_ref[rows, 1:2]
        m_new = jnp.maximum(m_old, m_c)
        p = w * jnp.exp(jnp.minimum(s - m_new, 0.0))
        scale = jnp.exp(m_old - m_new)

        st_ref[rows, 0:1] = m_new
        st_ref[rows, 1:2] = l_old * scale + jnp.sum(p, axis=1, keepdims=True)
        o_c = lax.dot_general(
            p.astype(jnp.bfloat16), v_all[:, h, :],
            dimension_numbers=(((1,), (0,)), ((), ())),
            preferred_element_type=jnp.float32,
        )
        o_ref[rows, :] = o_ref[rows, :] * scale + o_c


def _exchange_body(o_ref, s_ref, out_ref, comm_o, comm_s, send_sems, recv_sems):
    x = lax.axis_index("x")
    y = lax.axis_index("y")
    z = lax.axis_index("z")
    nbr = (x, y, 1 - z)

    barrier_sem = pltpu.get_barrier_semaphore()
    pl.semaphore_signal(
        barrier_sem, inc=1, device_id=nbr,
        device_id_type=pl.DeviceIdType.MESH,
    )
    pl.semaphore_wait(barrier_sem, 1)

    rdma_o = pltpu.make_async_remote_copy(
        src_ref=o_ref, dst_ref=comm_o,
        send_sem=send_sems.at[0], recv_sem=recv_sems.at[0],
        device_id=nbr, device_id_type=pl.DeviceIdType.MESH,
    )
    rdma_s = pltpu.make_async_remote_copy(
        src_ref=s_ref, dst_ref=comm_s,
        send_sem=send_sems.at[1], recv_sem=recv_sems.at[1],
        device_id=nbr, device_id_type=pl.DeviceIdType.MESH,
    )
    rdma_o.start()
    rdma_s.start()
    rdma_o.wait()
    rdma_s.wait()

    m0 = s_ref[:, 0:1]
    l0 = s_ref[:, 1:2]
    m1 = comm_s[:, 0:1]
    l1 = comm_s[:, 1:2]
    mg = jnp.maximum(m0, m1)
    a0 = jnp.exp(m0 - mg)
    a1 = jnp.exp(m1 - mg)
    lg = l0 * a0 + l1 * a1
    o = o_ref[...] * (a0 / lg) + comm_o[...] * (a1 / lg)
    out_ref[...] = jnp.transpose(o.reshape(H, B, D), (1, 0, 2)) \
        .reshape(B, 1, H, D)


def kernel(Q, K, V, bt, lens):
    n_local_pages = K.shape[0]
    chunk_pages = 16
    n_chunks = n_local_pages // chunk_pages

    qf = Q.reshape(B, H * D)
    lens2 = lens.reshape(B, 1)

    o_un, stats = pl.pallas_call(
        _compute_body,
        grid=(n_chunks,),
        in_specs=[
            pl.BlockSpec((B, H * D), lambda c: (0, 0)),
            pl.BlockSpec((chunk_pages, BS, H, D), lambda c: (c, 0, 0, 0)),
            pl.BlockSpec((chunk_pages, BS, H, D), lambda c: (c, 0, 0, 0)),
            pl.BlockSpec(bt.shape, lambda c: (0, 0)),
            pl.BlockSpec((B, 1), lambda c: (0, 0)),
        ],
        out_specs=[
            pl.BlockSpec((H * B, D), lambda c: (0, 0)),
            pl.BlockSpec((H * B, 8), lambda c: (0, 0)),
        ],
        out_shape=[
            jax.ShapeDtypeStruct((H * B, D), jnp.float32),
            jax.ShapeDtypeStruct((H * B, 8), jnp.float32),
        ],
    )(qf, K, V, bt, lens2)

    return pl.pallas_call(
        _exchange_body,
        out_shape=jax.ShapeDtypeStruct((B, 1, H, D), jnp.float32),
        in_specs=[
            pl.BlockSpec(memory_space=pltpu.VMEM),
            pl.BlockSpec(memory_space=pltpu.VMEM),
        ],
        out_specs=pl.BlockSpec(memory_space=pltpu.VMEM),
        scratch_shapes=[
            pltpu.VMEM((H * B, D), jnp.float32),
            pltpu.VMEM((H * B, 8), jnp.float32),
            pltpu.SemaphoreType.DMA((2,)),
            pltpu.SemaphoreType.DMA((2,)),
        ],
        compiler_params=pltpu.CompilerParams(collective_id=0),
    )(o_un, stats)


# device time: 57046 ns/iter; 4.9606x vs baseline; 2.3569x over previous
import functools

import jax
import jax.numpy as jnp
from jax import lax
from jax.experimental import pallas as pl
from jax.experimental.pallas import tpu as pltpu

B, H, D, BS = 32, 16, 128, 32
NEG = -1e30


def _body(q_ref, k_ref, v_ref, bt_ref, lens_ref, out_ref,
          o_acc, st_acc, o_send, st_send, comm_o, comm_s,
          o_send_sems, o_recv_sems, s_send_sems, s_recv_sems,
          *, n_chunks):
    c = pl.program_id(0)
    half = n_chunks // 2
    n_pages = k_ref.shape[0]
    t = n_pages * BS

    x = lax.axis_index("x")
    y = lax.axis_index("y")
    z = lax.axis_index("z")
    nbr = (x, y, 1 - z)

    def _rdma_pair(slot):
        ro = pltpu.make_async_remote_copy(
            src_ref=o_send.at[slot], dst_ref=comm_o.at[slot],
            send_sem=o_send_sems.at[slot], recv_sem=o_recv_sems.at[slot],
            device_id=nbr, device_id_type=pl.DeviceIdType.MESH,
        )
        rs = pltpu.make_async_remote_copy(
            src_ref=st_send.at[slot], dst_ref=comm_s.at[slot],
            send_sem=s_send_sems.at[slot], recv_sem=s_recv_sems.at[slot],
            device_id=nbr, device_id_type=pl.DeviceIdType.MESH,
        )
        return ro, rs

    @pl.when(jnp.logical_or(c == 0, c == half))
    def _():
        o_acc[...] = jnp.zeros_like(o_acc)
        st_acc[...] = jnp.zeros_like(st_acc)
        st_acc[0:H, :] = jnp.full((H, B), NEG, jnp.float32)

    @pl.when(c == 0)
    def _():
        barrier_sem = pltpu.get_barrier_semaphore()
        pl.semaphore_signal(
            barrier_sem, inc=1, device_id=nbr,
            device_id_type=pl.DeviceIdType.MESH,
        )
        pl.semaphore_wait(barrier_sem, 1)

    base = z * 256 + c * n_pages

    bt = bt_ref[...]
    nb = bt.shape[1]
    jidx = lax.broadcasted_iota(jnp.int32, bt.shape, 1)
    validf = (jidx < lens_ref[...]).astype(jnp.float32)
    bt3 = lax.broadcast_in_dim(bt, (B, n_pages, nb), (0, 2))
    valid3 = lax.broadcast_in_dim(validf, (B, n_pages, nb), (0, 2))
    pidx3 = lax.broadcasted_iota(jnp.int32, (B, n_pages, nb), 1) + base
    eqf = (bt3 == pidx3).astype(jnp.float32) * valid3
    counts = jnp.sum(eqf, axis=2)
    w = jnp.broadcast_to(counts[:, :, None], (B, n_pages, BS)).reshape(B, t)
    w3 = w[None, :, :]

    qb = (q_ref[...] * (D ** -0.5)).astype(jnp.bfloat16)
    k_t = jnp.transpose(
        k_ref[...].reshape(t, H, D).astype(jnp.bfloat16), (1, 0, 2))
    v_t = jnp.transpose(
        v_ref[...].reshape(t, H, D).astype(jnp.bfloat16), (1, 0, 2))

    s = lax.dot_general(
        qb, k_t,
        dimension_numbers=(((2,), (2,)), ((0,), (0,))),
        preferred_element_type=jnp.float32,
    )

    m_c = jnp.max(s, axis=2)
    m_old = st_acc[0:H, :]
    l_old = st_acc[H:2 * H, :]
    m_new = jnp.maximum(m_old, m_c)
    p = w3 * jnp.exp(s - m_new[:, :, None])
    scale = jnp.exp(m_old - m_new)

    st_acc[0:H, :] = m_new
    st_acc[H:2 * H, :] = l_old * scale + jnp.sum(p, axis=2)
    o_c = lax.dot_general(
        p.astype(jnp.bfloat16), v_t,
        dimension_numbers=(((2,), (1,)), ((0,), (0,))),
        preferred_element_type=jnp.float32,
    )
    o_acc[...] = o_acc[...] * scale[:, :, None] + o_c

    @pl.when(c == half - 1)
    def _():
        o_send[0] = o_acc[...].astype(jnp.bfloat16)
        st_send[0] = st_acc[...]
        ro, rs = _rdma_pair(0)
        ro.start()
        rs.start()

    @pl.when(c == n_chunks - 1)
    def _():
        o_send[1] = o_acc[...].astype(jnp.bfloat16)
        st_send[1] = st_acc[...]
        ro1, rs1 = _rdma_pair(1)
        ro1.start()
        rs1.start()
        ro0, rs0 = _rdma_pair(0)
        ro0.wait()
        rs0.wait()
        ro1.wait()
        rs1.wait()

        parts = [
            (o_send[0].astype(jnp.float32), st_send[0]),
            (o_acc[...], st_acc[...]),
            (comm_o[0].astype(jnp.float32), comm_s[0]),
            (comm_o[1].astype(jnp.float32), comm_s[1]),
        ]
        mg = functools.reduce(
            jnp.maximum, [st[0:H, :] for _, st in parts])
        lg = jnp.zeros_like(mg)
        o = jnp.zeros_like(parts[0][0])
        for op, st in parts:
            a = jnp.exp(st[0:H, :] - mg)
            lg = lg + st[H:2 * H, :] * a
            o = o + op * a[:, :, None]
        o = o / lg[:, :, None]
        out_ref[...] = jnp.transpose(o, (1, 0, 2)).reshape(B, 1, H, D)


def kernel(Q, K, V, bt, lens):
    n_local_pages = K.shape[0]
    chunk_pages = 16
    n_chunks = n_local_pages // chunk_pages

    qt = jnp.transpose(Q.reshape(B, H, D), (1, 0, 2))
    lens2 = lens.reshape(B, 1)

    return pl.pallas_call(
        functools.partial(_body, n_chunks=n_chunks),
        grid=(n_chunks,),
        in_specs=[
            pl.BlockSpec((H, B, D), lambda c: (0, 0, 0)),
            pl.BlockSpec((chunk_pages, BS, H, D), lambda c: (c, 0, 0, 0)),
            pl.BlockSpec((chunk_pages, BS, H, D), lambda c: (c, 0, 0, 0)),
            pl.BlockSpec(bt.shape, lambda c: (0, 0)),
            pl.BlockSpec((B, 1), lambda c: (0, 0)),
        ],
        out_specs=pl.BlockSpec((B, 1, H, D), lambda c: (0, 0, 0, 0)),
        out_shape=jax.ShapeDtypeStruct((B, 1, H, D), jnp.float32),
        scratch_shapes=[
            pltpu.VMEM((H, B, D), jnp.float32),
            pltpu.VMEM((2 * H, B), jnp.float32),
            pltpu.VMEM((2, H, B, D), jnp.bfloat16),
            pltpu.VMEM((2, 2 * H, B), jnp.float32),
            pltpu.VMEM((2, H, B, D), jnp.bfloat16),
            pltpu.VMEM((2, 2 * H, B), jnp.float32),
            pltpu.SemaphoreType.DMA((2,)),
            pltpu.SemaphoreType.DMA((2,)),
            pltpu.SemaphoreType.DMA((2,)),
            pltpu.SemaphoreType.DMA((2,)),
        ],
        compiler_params=pltpu.CompilerParams(collective_id=0),
    )(qt, K, V, bt, lens2)
